# baseline (speedup 1.0000x reference)
import dataclasses

import jax
import jax.numpy as jnp
from jax.experimental import pallas as pl
from jax.experimental.pallas import tpu as pltpu
from jax.experimental.pallas import tpu_sc as plsc

_DROP_RATIO = 0.2
_B_COEF = 1.0
_C_COEF = 1.0
_EPS = 1e-08

_CB = 8


def _sum_copy_kernel(x_ref, copy_ref, sums_ref):
    blk = x_ref[...]
    copy_ref[...] = blk
    sums_ref[...] = jnp.sum(blk, axis=(2, 3), keepdims=True)


def _sc_mask_kernel(n, c, k, kpad, sums_hbm, lanes_hbm, idx_hbm,
                    srow, lvm, sbuf2, tb, irow, sem):
    nv = c // 16
    core = jax.lax.axis_index("core")
    sub = jax.lax.axis_index("subcore")
    gg = sub * 2 + core
    g = jax.lax.div(gg, 4)
    part = jax.lax.rem(gg, 4)
    parts = [(0, 1), (2, 3), (4,), (5,)]

    def splat_sum(v):
        for r in (1, 2, 4, 8):
            tb[pl.ds(0, 16)] = v
            tb[pl.ds(16, 16)] = v
            v = v + tb[pl.ds(r, 16)]
        return v

    pltpu.async_copy(lanes_hbm, lvm, sem).wait()
    pltpu.async_copy(sums_hbm.at[g], srow, sem).wait()
    lane = lvm[...]
    izero = lane * 0
    ione = izero + 1
    fzero = lane.astype(jnp.float32) * 0.0
    fone = fzero + 1.0
    inv_hw = jnp.float32(1.0 / (224.0 * 224.0))
    xm = [srow[pl.ds(16 * j, 16)] * inv_hw for j in range(nv)]
    tot = xm[0]
    for j in range(1, nv):
        tot = tot + xm[j]
    sum_all = splat_sum(tot)
    sq = [v * v for v in xm]
    tot2 = sq[0]
    for j in range(1, nv):
        tot2 = tot2 + sq[j]
    x2_sum = splat_sum(tot2)
    denom = (1.0 + _B_COEF * x2_sum) * (1.0 + _B_COEF * x2_sum)
    scale = denom + _EPS
    s_vecs = [(sum_all - xm[j]) * sq[j] / scale for j in range(nv)]
    for j in range(nv):
        sbuf2[pl.ds(16 * j, 16)] = s_vecs[j]
        sbuf2[pl.ds(c + 16 * j, 16)] = s_vecs[j]

    def emit(my_js):
        ranks = {}
        for j in my_js:
            rk = izero
            for r in range(1, c):
                w = sbuf2[pl.ds(16 * j + r, 16)]
                gt = w > s_vecs[j]
                wrap = lane >= (c - r - 16 * j)
                eq = (w == s_vecs[j]) & wrap
                rk = rk + jnp.where(gt | eq, ione, izero)
            ranks[j] = rk
        out_vecs = [izero - 1 for _ in range(kpad // 16)]
        for s in range(k):
            acc = fzero
            cnt = fzero
            for j in my_js:
                hits = ranks[j] == s
                acc = acc + jnp.where(hits, (lane + 16 * j).astype(
                    jnp.float32), fzero)
                cnt = cnt + jnp.where(hits, fone, fzero)
            chan = splat_sum(acc).astype(jnp.int32)
            have = splat_sum(cnt) > 0.5
            t, l = divmod(s, 16)
            upd = jnp.where(have, chan, izero - 1)
            out_vecs[t] = jnp.where(lane == l, upd, out_vecs[t])
        for t in range(kpad // 16):
            irow[pl.ds(16 * t, 16)] = out_vecs[t]

    for p, my_js in enumerate(parts):
        @pl.when(part == p)
        def _(my_js=my_js):
            emit(my_js)
    pltpu.async_copy(irow, idx_hbm.at[gg], sem).wait()


def _mask_kernel(k, kpad, sums_ref, idx_ref):
    n, c = sums_ref.shape
    hw = jnp.float32(224 * 224)
    xm = sums_ref[...] / hw
    x2_sum = jnp.sum(xm * xm, axis=1, keepdims=True)
    sum_all = jnp.sum(xm, axis=1, keepdims=True)
    neighbor = sum_all - xm
    denom = (1.0 + _B_COEF * x2_sum) * (1.0 + _B_COEF * x2_sum)
    s = neighbor * (xm * xm) / (denom + _EPS)
    ci = jax.lax.broadcasted_iota(jnp.int32, (n, c), 1)
    a = s[:, None, :]
    b = s[:, :, None]
    gt = jnp.sum((a > b).astype(jnp.int32), axis=2)
    eql = jnp.sum(
        ((a == b) & (ci[:, None, :] < ci[:, :, None])).astype(jnp.int32),
        axis=2,
    )
    rank = gt + eql
    jj = jax.lax.broadcasted_iota(jnp.int32, (n, kpad, c), 1)
    hits = (rank[:, None, :] == jj).astype(jnp.int32)
    idx_ref[...] = jnp.sum(hits * ci[:, None, :], axis=2)


def _zero_kernel(nk, c, idx_ref, x_ref, out_ref, zeros_ref, sem):
    del x_ref
    zeros_ref[...] = jnp.zeros_like(zeros_ref)

    def mk(i):
        row = idx_ref[i]
        nn = jax.lax.div(row, c)
        cc = jax.lax.rem(row, c)
        return pltpu.make_async_copy(
            zeros_ref, out_ref.at[pl.ds(nn, 1), pl.ds(cc, 1)], sem)

    def start(i, _):
        mk(i).start()
        return 0

    jax.lax.fori_loop(0, nk, start, 0)

    def wait(i, _):
        mk(i).wait()
        return 0

    jax.lax.fori_loop(0, nk, wait, 0)


def kernel(x):
    n, c, h, w = x.shape
    k = max(1, int(round(_DROP_RATIO * c)))
    kpad = 32

    copy, sums = pl.pallas_call(
        _sum_copy_kernel,
        grid=(n, c // _CB),
        in_specs=[pl.BlockSpec((1, _CB, h, w), lambda i, j: (i, j, 0, 0))],
        out_specs=[
            pl.BlockSpec((1, _CB, h, w), lambda i, j: (i, j, 0, 0)),
            pl.BlockSpec((1, _CB, 1, 1), lambda i, j: (i, j, 0, 0)),
        ],
        out_shape=[
            jax.ShapeDtypeStruct((n, c, h, w), x.dtype),
            jax.ShapeDtypeStruct((n, c, 1, 1), jnp.float32),
        ],
    )(x)

    sums_nc = sums.reshape(n, c)
    sc_mesh = plsc.VectorSubcoreMesh(core_axis_name="core",
                                     subcore_axis_name="subcore")
    lanes = jnp.arange(16, dtype=jnp.int32)
    idx = pl.kernel(
        lambda s_hbm, l_hbm, i_hbm, srow, lvm, sbuf2, tb, irow, sem:
            _sc_mask_kernel(n, c, k, kpad, s_hbm, l_hbm, i_hbm,
                            srow, lvm, sbuf2, tb, irow, sem),
        out_type=jax.ShapeDtypeStruct((4 * n, kpad), jnp.int32),
        mesh=sc_mesh,
        scratch_types=[
            pltpu.VMEM((c,), jnp.float32),
            pltpu.VMEM((16,), jnp.int32),
            pltpu.VMEM((2 * c,), jnp.float32),
            pltpu.VMEM((32,), jnp.float32),
            pltpu.VMEM((kpad,), jnp.int32),
            pltpu.SemaphoreType.DMA,
        ],
    )(sums_nc, lanes)
    idx_nc = idx.reshape(n, 4, kpad).max(axis=1)
    drop_rows = (idx_nc[:, :k] + jnp.arange(n, dtype=jnp.int32)[:, None] * c
                 ).reshape(n * k)

    out = pl.pallas_call(
        lambda i_ref, x_ref, o_ref, z_ref, sem: _zero_kernel(
            n * k, c, i_ref, x_ref, o_ref, z_ref, sem),
        grid_spec=pltpu.PrefetchScalarGridSpec(
            num_scalar_prefetch=1,
            grid=(1,),
            in_specs=[pl.BlockSpec(memory_space=pl.ANY)],
            out_specs=pl.BlockSpec(memory_space=pl.ANY),
            scratch_shapes=[
                pltpu.VMEM((1, 1, h, w), x.dtype),
                pltpu.SemaphoreType.DMA,
            ],
        ),
        out_shape=jax.ShapeDtypeStruct((n, c, h, w), x.dtype),
        input_output_aliases={1: 0},
    )(drop_rows, copy)

    return out


# speedup vs baseline: 1.0276x; 1.0276x over previous
import dataclasses

import jax
import jax.numpy as jnp
from jax.experimental import pallas as pl
from jax.experimental.pallas import tpu as pltpu
from jax.experimental.pallas import tpu_sc as plsc

_DROP_RATIO = 0.2
_B_COEF = 1.0
_C_COEF = 1.0
_EPS = 1e-08

_CB = 8


def _sum_copy_kernel(x_ref, copy_ref, sums_ref):
    blk = x_ref[...]
    copy_ref[...] = blk
    sums_ref[...] = jnp.sum(blk, axis=(2, 3), keepdims=True)


def _sc_mask_kernel(n, c, k, kpad, sums_hbm, lanes_hbm, idx_hbm,
                    srow, lvm, sbuf2, tb, irow, sem):
    nv = c // 16
    core = jax.lax.axis_index("core")
    sub = jax.lax.axis_index("subcore")
    g = sub * 2 + core

    def splat_sum(v):
        for r in (1, 2, 4, 8):
            tb[pl.ds(0, 16)] = v
            tb[pl.ds(16, 16)] = v
            v = v + tb[pl.ds(r, 16)]
        return v

    @pl.when(g < n)
    def _():
        pltpu.async_copy(lanes_hbm, lvm, sem).wait()
        pltpu.async_copy(sums_hbm.at[g], srow, sem).wait()
        lane = lvm[...]
        izero = lane * 0
        ione = izero + 1
        fzero = lane.astype(jnp.float32) * 0.0
        inv_hw = jnp.float32(1.0 / (224.0 * 224.0))
        xm = [srow[pl.ds(16 * j, 16)] * inv_hw for j in range(nv)]
        tot = xm[0]
        for j in range(1, nv):
            tot = tot + xm[j]
        sum_all = splat_sum(tot)
        sq = [v * v for v in xm]
        tot2 = sq[0]
        for j in range(1, nv):
            tot2 = tot2 + sq[j]
        x2_sum = splat_sum(tot2)
        denom = (1.0 + _B_COEF * x2_sum) * (1.0 + _B_COEF * x2_sum)
        scale = denom + _EPS
        s_vecs = [(sum_all - xm[j]) * sq[j] / scale for j in range(nv)]
        for j in range(nv):
            sbuf2[pl.ds(16 * j, 16)] = s_vecs[j]
            sbuf2[pl.ds(c + 16 * j, 16)] = s_vecs[j]
        ranks = [izero for _ in range(nv)]
        for r in range(1, c):
            for j in range(nv):
                w = sbuf2[pl.ds(16 * j + r, 16)]
                gt = w > s_vecs[j]
                wrap = lane >= (c - r - 16 * j)
                eq = (w == s_vecs[j]) & wrap
                ranks[j] = ranks[j] + jnp.where(gt | eq, ione, izero)
        out_vecs = [izero for _ in range(kpad // 16)]
        for s in range(k):
            acc = fzero
            for j in range(nv):
                hits = ranks[j] == s
                acc = acc + jnp.where(hits, (lane + 16 * j).astype(
                    jnp.float32), fzero)
            chan = splat_sum(acc).astype(jnp.int32) + g * c
            t, l = divmod(s, 16)
            out_vecs[t] = out_vecs[t] + jnp.where(lane == l, chan, izero)
        for t in range(kpad // 16):
            irow[pl.ds(16 * t, 16)] = out_vecs[t]
        pltpu.async_copy(irow, idx_hbm.at[g], sem).wait()


def _mask_kernel(k, kpad, sums_ref, idx_ref):
    n, c = sums_ref.shape
    hw = jnp.float32(224 * 224)
    xm = sums_ref[...] / hw
    x2_sum = jnp.sum(xm * xm, axis=1, keepdims=True)
    sum_all = jnp.sum(xm, axis=1, keepdims=True)
    neighbor = sum_all - xm
    denom = (1.0 + _B_COEF * x2_sum) * (1.0 + _B_COEF * x2_sum)
    s = neighbor * (xm * xm) / (denom + _EPS)
    ci = jax.lax.broadcasted_iota(jnp.int32, (n, c), 1)
    a = s[:, None, :]
    b = s[:, :, None]
    gt = jnp.sum((a > b).astype(jnp.int32), axis=2)
    eql = jnp.sum(
        ((a == b) & (ci[:, None, :] < ci[:, :, None])).astype(jnp.int32),
        axis=2,
    )
    rank = gt + eql
    jj = jax.lax.broadcasted_iota(jnp.int32, (n, kpad, c), 1)
    hits = (rank[:, None, :] == jj).astype(jnp.int32)
    idx_ref[...] = jnp.sum(hits * ci[:, None, :], axis=2)


def _zero_kernel(nk, c, k, idx_ref, x_ref, out_ref, zeros_ref, sem):
    del x_ref
    zeros_ref[...] = jnp.zeros_like(zeros_ref)

    def mk(i):
        row = idx_ref[jax.lax.div(i, k), jax.lax.rem(i, k)]
        nn = jax.lax.div(row, c)
        cc = jax.lax.rem(row, c)
        return pltpu.make_async_copy(
            zeros_ref, out_ref.at[pl.ds(nn, 1), pl.ds(cc, 1)], sem)

    def start(i, _):
        mk(i).start()
        return 0

    jax.lax.fori_loop(0, nk, start, 0)

    def wait(i, _):
        mk(i).wait()
        return 0

    jax.lax.fori_loop(0, nk, wait, 0)


def kernel(x):
    n, c, h, w = x.shape
    k = max(1, int(round(_DROP_RATIO * c)))
    kpad = 32

    copy, sums = pl.pallas_call(
        _sum_copy_kernel,
        grid=(n, c // _CB),
        in_specs=[pl.BlockSpec((1, _CB, h, w), lambda i, j: (i, j, 0, 0))],
        out_specs=[
            pl.BlockSpec((1, _CB, h, w), lambda i, j: (i, j, 0, 0)),
            pl.BlockSpec((1, _CB, 1, 1), lambda i, j: (i, j, 0, 0)),
        ],
        out_shape=[
            jax.ShapeDtypeStruct((n, c, h, w), x.dtype),
            jax.ShapeDtypeStruct((n, c, 1, 1), jnp.float32),
        ],
    )(x)

    sums_nc = sums.reshape(n, c)
    sc_mesh = plsc.VectorSubcoreMesh(core_axis_name="core",
                                     subcore_axis_name="subcore")
    lanes = jnp.arange(16, dtype=jnp.int32)
    idx = pl.kernel(
        lambda s_hbm, l_hbm, i_hbm, srow, lvm, sbuf2, tb, irow, sem:
            _sc_mask_kernel(n, c, k, kpad, s_hbm, l_hbm, i_hbm,
                            srow, lvm, sbuf2, tb, irow, sem),
        out_type=jax.ShapeDtypeStruct((n, kpad), jnp.int32),
        mesh=sc_mesh,
        scratch_types=[
            pltpu.VMEM((c,), jnp.float32),
            pltpu.VMEM((16,), jnp.int32),
            pltpu.VMEM((2 * c,), jnp.float32),
            pltpu.VMEM((32,), jnp.float32),
            pltpu.VMEM((kpad,), jnp.int32),
            pltpu.SemaphoreType.DMA,
        ],
    )(sums_nc, lanes)

    out = pl.pallas_call(
        lambda i_ref, x_ref, o_ref, z_ref, sem: _zero_kernel(
            n * k, c, k, i_ref, x_ref, o_ref, z_ref, sem),
        grid_spec=pltpu.PrefetchScalarGridSpec(
            num_scalar_prefetch=1,
            grid=(1,),
            in_specs=[pl.BlockSpec(memory_space=pl.ANY)],
            out_specs=pl.BlockSpec(memory_space=pl.ANY),
            scratch_shapes=[
                pltpu.VMEM((1, 1, h, w), x.dtype),
                pltpu.SemaphoreType.DMA,
            ],
        ),
        out_shape=jax.ShapeDtypeStruct((n, c, h, w), x.dtype),
        input_output_aliases={1: 0},
    )(idx, copy)

    return out


# speedup vs baseline: 1.1390x; 1.1084x over previous
import dataclasses

import jax
import jax.numpy as jnp
from jax.experimental import pallas as pl
from jax.experimental.pallas import tpu as pltpu
from jax.experimental.pallas import tpu_sc as plsc

_DROP_RATIO = 0.2
_B_COEF = 1.0
_C_COEF = 1.0
_EPS = 1e-08

_CB = 16


def _sum_copy_kernel(x_ref, copy_ref, sums_ref):
    blk = x_ref[...]
    copy_ref[...] = blk
    sums_ref[...] = jnp.sum(blk, axis=(2, 3), keepdims=True)


def _sc_mask_kernel(n, c, k, kpad, sums_hbm, lanes_hbm, idx_hbm,
                    srow, lvm, sbuf2, tb, irow, sem):
    nv = c // 16
    core = jax.lax.axis_index("core")
    sub = jax.lax.axis_index("subcore")
    g = sub * 2 + core

    def splat_sum(v):
        for r in (1, 2, 4, 8):
            tb[pl.ds(0, 16)] = v
            tb[pl.ds(16, 16)] = v
            v = v + tb[pl.ds(r, 16)]
        return v

    @pl.when(g < n)
    def _():
        pltpu.async_copy(lanes_hbm, lvm, sem).wait()
        pltpu.async_copy(sums_hbm.at[g], srow, sem).wait()
        lane = lvm[...]
        izero = lane * 0
        ione = izero + 1
        fzero = lane.astype(jnp.float32) * 0.0
        inv_hw = jnp.float32(1.0 / (224.0 * 224.0))
        xm = [srow[pl.ds(16 * j, 16)] * inv_hw for j in range(nv)]
        tot = xm[0]
        for j in range(1, nv):
            tot = tot + xm[j]
        sum_all = splat_sum(tot)
        sq = [v * v for v in xm]
        tot2 = sq[0]
        for j in range(1, nv):
            tot2 = tot2 + sq[j]
        x2_sum = splat_sum(tot2)
        denom = (1.0 + _B_COEF * x2_sum) * (1.0 + _B_COEF * x2_sum)
        scale = denom + _EPS
        s_vecs = [(sum_all - xm[j]) * sq[j] / scale for j in range(nv)]
        for j in range(nv):
            sbuf2[pl.ds(16 * j, 16)] = s_vecs[j]
            sbuf2[pl.ds(c + 16 * j, 16)] = s_vecs[j]
        ranks = [izero for _ in range(nv)]
        for r in range(1, c):
            for j in range(nv):
                w = sbuf2[pl.ds(16 * j + r, 16)]
                gt = w > s_vecs[j]
                wrap = lane >= (c - r - 16 * j)
                eq = (w == s_vecs[j]) & wrap
                ranks[j] = ranks[j] + jnp.where(gt | eq, ione, izero)
        out_vecs = [izero for _ in range(kpad // 16)]
        for s in range(k):
            acc = fzero
            for j in range(nv):
                hits = ranks[j] == s
                acc = acc + jnp.where(hits, (lane + 16 * j).astype(
                    jnp.float32), fzero)
            chan = splat_sum(acc).astype(jnp.int32) + g * c
            t, l = divmod(s, 16)
            out_vecs[t] = out_vecs[t] + jnp.where(lane == l, chan, izero)
        for t in range(kpad // 16):
            irow[pl.ds(16 * t, 16)] = out_vecs[t]
        pltpu.async_copy(irow, idx_hbm.at[g], sem).wait()


def _mask_kernel(k, kpad, sums_ref, idx_ref):
    n, c = sums_ref.shape
    hw = jnp.float32(224 * 224)
    xm = sums_ref[...] / hw
    x2_sum = jnp.sum(xm * xm, axis=1, keepdims=True)
    sum_all = jnp.sum(xm, axis=1, keepdims=True)
    neighbor = sum_all - xm
    denom = (1.0 + _B_COEF * x2_sum) * (1.0 + _B_COEF * x2_sum)
    s = neighbor * (xm * xm) / (denom + _EPS)
    ci = jax.lax.broadcasted_iota(jnp.int32, (n, c), 1)
    a = s[:, None, :]
    b = s[:, :, None]
    gt = jnp.sum((a > b).astype(jnp.int32), axis=2)
    eql = jnp.sum(
        ((a == b) & (ci[:, None, :] < ci[:, :, None])).astype(jnp.int32),
        axis=2,
    )
    rank = gt + eql
    jj = jax.lax.broadcasted_iota(jnp.int32, (n, kpad, c), 1)
    hits = (rank[:, None, :] == jj).astype(jnp.int32)
    idx_ref[...] = jnp.sum(hits * ci[:, None, :], axis=2)


def _zero_kernel(nk, c, k, idx_ref, x_ref, out_ref, zeros_ref, sem):
    del x_ref
    zeros_ref[...] = jnp.zeros_like(zeros_ref)

    def mk(i):
        row = idx_ref[jax.lax.div(i, k), jax.lax.rem(i, k)]
        nn = jax.lax.div(row, c)
        cc = jax.lax.rem(row, c)
        return pltpu.make_async_copy(
            zeros_ref, out_ref.at[pl.ds(nn, 1), pl.ds(cc, 1)], sem)

    def start(i, _):
        mk(i).start()
        return 0

    jax.lax.fori_loop(0, nk, start, 0)

    def wait(i, _):
        mk(i).wait()
        return 0

    jax.lax.fori_loop(0, nk, wait, 0)


def kernel(x):
    n, c, h, w = x.shape
    k = max(1, int(round(_DROP_RATIO * c)))
    kpad = 32

    copy, sums = pl.pallas_call(
        _sum_copy_kernel,
        grid=(n, c // _CB),
        in_specs=[pl.BlockSpec((1, _CB, h, w), lambda i, j: (i, j, 0, 0))],
        out_specs=[
            pl.BlockSpec((1, _CB, h, w), lambda i, j: (i, j, 0, 0)),
            pl.BlockSpec((1, _CB, 1, 1), lambda i, j: (i, j, 0, 0)),
        ],
        out_shape=[
            jax.ShapeDtypeStruct((n, c, h, w), x.dtype),
            jax.ShapeDtypeStruct((n, c, 1, 1), jnp.float32),
        ],
    )(x)

    sums_nc = sums.reshape(n, c)
    sc_mesh = plsc.VectorSubcoreMesh(core_axis_name="core",
                                     subcore_axis_name="subcore")
    lanes = jnp.arange(16, dtype=jnp.int32)
    idx = pl.kernel(
        lambda s_hbm, l_hbm, i_hbm, srow, lvm, sbuf2, tb, irow, sem:
            _sc_mask_kernel(n, c, k, kpad, s_hbm, l_hbm, i_hbm,
                            srow, lvm, sbuf2, tb, irow, sem),
        out_type=jax.ShapeDtypeStruct((n, kpad), jnp.int32),
        mesh=sc_mesh,
        scratch_types=[
            pltpu.VMEM((c,), jnp.float32),
            pltpu.VMEM((16,), jnp.int32),
            pltpu.VMEM((2 * c,), jnp.float32),
            pltpu.VMEM((32,), jnp.float32),
            pltpu.VMEM((kpad,), jnp.int32),
            pltpu.SemaphoreType.DMA,
        ],
    )(sums_nc, lanes)

    out = pl.pallas_call(
        lambda i_ref, x_ref, o_ref, z_ref, sem: _zero_kernel(
            n * k, c, k, i_ref, x_ref, o_ref, z_ref, sem),
        grid_spec=pltpu.PrefetchScalarGridSpec(
            num_scalar_prefetch=1,
            grid=(1,),
            in_specs=[pl.BlockSpec(memory_space=pl.ANY)],
            out_specs=pl.BlockSpec(memory_space=pl.ANY),
            scratch_shapes=[
                pltpu.VMEM((1, 1, h, w), x.dtype),
                pltpu.SemaphoreType.DMA,
            ],
        ),
        out_shape=jax.ShapeDtypeStruct((n, c, h, w), x.dtype),
        input_output_aliases={1: 0},
    )(idx, copy)

    return out


# speedup vs baseline: 1.1630x; 1.0211x over previous
import dataclasses

import jax
import jax.numpy as jnp
from jax.experimental import pallas as pl
from jax.experimental.pallas import tpu as pltpu
from jax.experimental.pallas import tpu_sc as plsc

_DROP_RATIO = 0.2
_B_COEF = 1.0
_C_COEF = 1.0
_EPS = 1e-08

_CB = 32


def _sum_copy_kernel(x_ref, copy_ref, sums_ref):
    blk = x_ref[...]
    copy_ref[...] = blk
    sums_ref[...] = jnp.sum(blk, axis=(2, 3), keepdims=True)


def _sc_mask_kernel(n, c, k, kpad, sums_hbm, lanes_hbm, idx_hbm,
                    srow, lvm, sbuf2, tb, irow, sem):
    nv = c // 16
    core = jax.lax.axis_index("core")
    sub = jax.lax.axis_index("subcore")
    g = sub * 2 + core

    def splat_sum(v):
        for r in (1, 2, 4, 8):
            tb[pl.ds(0, 16)] = v
            tb[pl.ds(16, 16)] = v
            v = v + tb[pl.ds(r, 16)]
        return v

    @pl.when(g < n)
    def _():
        pltpu.async_copy(lanes_hbm, lvm, sem).wait()
        pltpu.async_copy(sums_hbm.at[g], srow, sem).wait()
        lane = lvm[...]
        izero = lane * 0
        ione = izero + 1
        fzero = lane.astype(jnp.float32) * 0.0
        inv_hw = jnp.float32(1.0 / (224.0 * 224.0))
        xm = [srow[pl.ds(16 * j, 16)] * inv_hw for j in range(nv)]
        tot = xm[0]
        for j in range(1, nv):
            tot = tot + xm[j]
        sum_all = splat_sum(tot)
        sq = [v * v for v in xm]
        tot2 = sq[0]
        for j in range(1, nv):
            tot2 = tot2 + sq[j]
        x2_sum = splat_sum(tot2)
        denom = (1.0 + _B_COEF * x2_sum) * (1.0 + _B_COEF * x2_sum)
        scale = denom + _EPS
        s_vecs = [(sum_all - xm[j]) * sq[j] / scale for j in range(nv)]
        for j in range(nv):
            sbuf2[pl.ds(16 * j, 16)] = s_vecs[j]
            sbuf2[pl.ds(c + 16 * j, 16)] = s_vecs[j]
        ranks = [izero for _ in range(nv)]
        for r in range(1, c):
            for j in range(nv):
                w = sbuf2[pl.ds(16 * j + r, 16)]
                gt = w > s_vecs[j]
                wrap = lane >= (c - r - 16 * j)
                eq = (w == s_vecs[j]) & wrap
                ranks[j] = ranks[j] + jnp.where(gt | eq, ione, izero)
        out_vecs = [izero for _ in range(kpad // 16)]
        for s in range(k):
            acc = fzero
            for j in range(nv):
                hits = ranks[j] == s
                acc = acc + jnp.where(hits, (lane + 16 * j).astype(
                    jnp.float32), fzero)
            chan = splat_sum(acc).astype(jnp.int32) + g * c
            t, l = divmod(s, 16)
            out_vecs[t] = out_vecs[t] + jnp.where(lane == l, chan, izero)
        for t in range(kpad // 16):
            irow[pl.ds(16 * t, 16)] = out_vecs[t]
        pltpu.async_copy(irow, idx_hbm.at[g], sem).wait()


def _mask_kernel(k, kpad, sums_ref, idx_ref):
    n, c = sums_ref.shape
    hw = jnp.float32(224 * 224)
    xm = sums_ref[...] / hw
    x2_sum = jnp.sum(xm * xm, axis=1, keepdims=True)
    sum_all = jnp.sum(xm, axis=1, keepdims=True)
    neighbor = sum_all - xm
    denom = (1.0 + _B_COEF * x2_sum) * (1.0 + _B_COEF * x2_sum)
    s = neighbor * (xm * xm) / (denom + _EPS)
    ci = jax.lax.broadcasted_iota(jnp.int32, (n, c), 1)
    a = s[:, None, :]
    b = s[:, :, None]
    gt = jnp.sum((a > b).astype(jnp.int32), axis=2)
    eql = jnp.sum(
        ((a == b) & (ci[:, None, :] < ci[:, :, None])).astype(jnp.int32),
        axis=2,
    )
    rank = gt + eql
    jj = jax.lax.broadcasted_iota(jnp.int32, (n, kpad, c), 1)
    hits = (rank[:, None, :] == jj).astype(jnp.int32)
    idx_ref[...] = jnp.sum(hits * ci[:, None, :], axis=2)


def _zero_kernel(nk, c, k, idx_ref, x_ref, out_ref, zeros_ref, sem):
    del x_ref
    zeros_ref[...] = jnp.zeros_like(zeros_ref)

    def mk(i):
        row = idx_ref[jax.lax.div(i, k), jax.lax.rem(i, k)]
        nn = jax.lax.div(row, c)
        cc = jax.lax.rem(row, c)
        return pltpu.make_async_copy(
            zeros_ref, out_ref.at[pl.ds(nn, 1), pl.ds(cc, 1)], sem)

    def start(i, _):
        mk(i).start()
        return 0

    jax.lax.fori_loop(0, nk, start, 0)

    def wait(i, _):
        mk(i).wait()
        return 0

    jax.lax.fori_loop(0, nk, wait, 0)


def kernel(x):
    n, c, h, w = x.shape
    k = max(1, int(round(_DROP_RATIO * c)))
    kpad = 32

    copy, sums = pl.pallas_call(
        _sum_copy_kernel,
        grid=(n, c // _CB),
        in_specs=[pl.BlockSpec((1, _CB, h, w), lambda i, j: (i, j, 0, 0))],
        out_specs=[
            pl.BlockSpec((1, _CB, h, w), lambda i, j: (i, j, 0, 0)),
            pl.BlockSpec((1, _CB, 1, 1), lambda i, j: (i, j, 0, 0)),
        ],
        out_shape=[
            jax.ShapeDtypeStruct((n, c, h, w), x.dtype),
            jax.ShapeDtypeStruct((n, c, 1, 1), jnp.float32),
        ],
    )(x)

    sums_nc = sums.reshape(n, c)
    sc_mesh = plsc.VectorSubcoreMesh(core_axis_name="core",
                                     subcore_axis_name="subcore")
    lanes = jnp.arange(16, dtype=jnp.int32)
    idx = pl.kernel(
        lambda s_hbm, l_hbm, i_hbm, srow, lvm, sbuf2, tb, irow, sem:
            _sc_mask_kernel(n, c, k, kpad, s_hbm, l_hbm, i_hbm,
                            srow, lvm, sbuf2, tb, irow, sem),
        out_type=jax.ShapeDtypeStruct((n, kpad), jnp.int32),
        mesh=sc_mesh,
        scratch_types=[
            pltpu.VMEM((c,), jnp.float32),
            pltpu.VMEM((16,), jnp.int32),
            pltpu.VMEM((2 * c,), jnp.float32),
            pltpu.VMEM((32,), jnp.float32),
            pltpu.VMEM((kpad,), jnp.int32),
            pltpu.SemaphoreType.DMA,
        ],
    )(sums_nc, lanes)

    out = pl.pallas_call(
        lambda i_ref, x_ref, o_ref, z_ref, sem: _zero_kernel(
            n * k, c, k, i_ref, x_ref, o_ref, z_ref, sem),
        grid_spec=pltpu.PrefetchScalarGridSpec(
            num_scalar_prefetch=1,
            grid=(1,),
            in_specs=[pl.BlockSpec(memory_space=pl.ANY)],
            out_specs=pl.BlockSpec(memory_space=pl.ANY),
            scratch_shapes=[
                pltpu.VMEM((1, 1, h, w), x.dtype),
                pltpu.SemaphoreType.DMA,
            ],
        ),
        out_shape=jax.ShapeDtypeStruct((n, c, h, w), x.dtype),
        input_output_aliases={1: 0},
    )(idx, copy)

    return out


# speedup vs baseline: 1.1708x; 1.0067x over previous
import dataclasses

import jax
import jax.numpy as jnp
from jax.experimental import pallas as pl
from jax.experimental.pallas import tpu as pltpu
from jax.experimental.pallas import tpu_sc as plsc

_DROP_RATIO = 0.2
_B_COEF = 1.0
_C_COEF = 1.0
_EPS = 1e-08

_CB = 48


def _sum_copy_kernel(x_ref, copy_ref, sums_ref):
    blk = x_ref[...]
    copy_ref[...] = blk
    sums_ref[...] = jnp.sum(blk, axis=(2, 3), keepdims=True)


def _sc_mask_kernel(n, c, k, kpad, sums_hbm, lanes_hbm, idx_hbm,
                    srow, lvm, sbuf2, tb, irow, sem):
    nv = c // 16
    core = jax.lax.axis_index("core")
    sub = jax.lax.axis_index("subcore")
    g = sub * 2 + core

    def splat_sum(v):
        for r in (1, 2, 4, 8):
            tb[pl.ds(0, 16)] = v
            tb[pl.ds(16, 16)] = v
            v = v + tb[pl.ds(r, 16)]
        return v

    @pl.when(g < n)
    def _():
        pltpu.async_copy(lanes_hbm, lvm, sem).wait()
        pltpu.async_copy(sums_hbm.at[g], srow, sem).wait()
        lane = lvm[...]
        izero = lane * 0
        ione = izero + 1
        fzero = lane.astype(jnp.float32) * 0.0
        inv_hw = jnp.float32(1.0 / (224.0 * 224.0))
        xm = [srow[pl.ds(16 * j, 16)] * inv_hw for j in range(nv)]
        tot = xm[0]
        for j in range(1, nv):
            tot = tot + xm[j]
        sum_all = splat_sum(tot)
        sq = [v * v for v in xm]
        tot2 = sq[0]
        for j in range(1, nv):
            tot2 = tot2 + sq[j]
        x2_sum = splat_sum(tot2)
        denom = (1.0 + _B_COEF * x2_sum) * (1.0 + _B_COEF * x2_sum)
        scale = denom + _EPS
        s_vecs = [(sum_all - xm[j]) * sq[j] / scale for j in range(nv)]
        for j in range(nv):
            sbuf2[pl.ds(16 * j, 16)] = s_vecs[j]
            sbuf2[pl.ds(c + 16 * j, 16)] = s_vecs[j]
        ranks = [izero for _ in range(nv)]
        for r in range(1, c):
            for j in range(nv):
                w = sbuf2[pl.ds(16 * j + r, 16)]
                gt = w > s_vecs[j]
                wrap = lane >= (c - r - 16 * j)
                eq = (w == s_vecs[j]) & wrap
                ranks[j] = ranks[j] + jnp.where(gt | eq, ione, izero)
        out_vecs = [izero for _ in range(kpad // 16)]
        for s in range(k):
            acc = fzero
            for j in range(nv):
                hits = ranks[j] == s
                acc = acc + jnp.where(hits, (lane + 16 * j).astype(
                    jnp.float32), fzero)
            chan = splat_sum(acc).astype(jnp.int32) + g * c
            t, l = divmod(s, 16)
            out_vecs[t] = out_vecs[t] + jnp.where(lane == l, chan, izero)
        for t in range(kpad // 16):
            irow[pl.ds(16 * t, 16)] = out_vecs[t]
        pltpu.async_copy(irow, idx_hbm.at[g], sem).wait()


def _mask_kernel(k, kpad, sums_ref, idx_ref):
    n, c = sums_ref.shape
    hw = jnp.float32(224 * 224)
    xm = sums_ref[...] / hw
    x2_sum = jnp.sum(xm * xm, axis=1, keepdims=True)
    sum_all = jnp.sum(xm, axis=1, keepdims=True)
    neighbor = sum_all - xm
    denom = (1.0 + _B_COEF * x2_sum) * (1.0 + _B_COEF * x2_sum)
    s = neighbor * (xm * xm) / (denom + _EPS)
    ci = jax.lax.broadcasted_iota(jnp.int32, (n, c), 1)
    a = s[:, None, :]
    b = s[:, :, None]
    gt = jnp.sum((a > b).astype(jnp.int32), axis=2)
    eql = jnp.sum(
        ((a == b) & (ci[:, None, :] < ci[:, :, None])).astype(jnp.int32),
        axis=2,
    )
    rank = gt + eql
    jj = jax.lax.broadcasted_iota(jnp.int32, (n, kpad, c), 1)
    hits = (rank[:, None, :] == jj).astype(jnp.int32)
    idx_ref[...] = jnp.sum(hits * ci[:, None, :], axis=2)


def _zero_kernel(nk, c, k, idx_ref, x_ref, out_ref, zeros_ref, sem):
    del x_ref
    zeros_ref[...] = jnp.zeros_like(zeros_ref)

    def mk(i):
        row = idx_ref[jax.lax.div(i, k), jax.lax.rem(i, k)]
        nn = jax.lax.div(row, c)
        cc = jax.lax.rem(row, c)
        return pltpu.make_async_copy(
            zeros_ref, out_ref.at[pl.ds(nn, 1), pl.ds(cc, 1)], sem)

    def start(i, _):
        mk(i).start()
        return 0

    jax.lax.fori_loop(0, nk, start, 0)

    def wait(i, _):
        mk(i).wait()
        return 0

    jax.lax.fori_loop(0, nk, wait, 0)


def kernel(x):
    n, c, h, w = x.shape
    k = max(1, int(round(_DROP_RATIO * c)))
    kpad = 32

    copy, sums = pl.pallas_call(
        _sum_copy_kernel,
        grid=(n, c // _CB),
        in_specs=[pl.BlockSpec((1, _CB, h, w), lambda i, j: (i, j, 0, 0))],
        out_specs=[
            pl.BlockSpec((1, _CB, h, w), lambda i, j: (i, j, 0, 0)),
            pl.BlockSpec((1, _CB, 1, 1), lambda i, j: (i, j, 0, 0)),
        ],
        out_shape=[
            jax.ShapeDtypeStruct((n, c, h, w), x.dtype),
            jax.ShapeDtypeStruct((n, c, 1, 1), jnp.float32),
        ],
    )(x)

    sums_nc = sums.reshape(n, c)
    sc_mesh = plsc.VectorSubcoreMesh(core_axis_name="core",
                                     subcore_axis_name="subcore")
    lanes = jnp.arange(16, dtype=jnp.int32)
    idx = pl.kernel(
        lambda s_hbm, l_hbm, i_hbm, srow, lvm, sbuf2, tb, irow, sem:
            _sc_mask_kernel(n, c, k, kpad, s_hbm, l_hbm, i_hbm,
                            srow, lvm, sbuf2, tb, irow, sem),
        out_type=jax.ShapeDtypeStruct((n, kpad), jnp.int32),
        mesh=sc_mesh,
        scratch_types=[
            pltpu.VMEM((c,), jnp.float32),
            pltpu.VMEM((16,), jnp.int32),
            pltpu.VMEM((2 * c,), jnp.float32),
            pltpu.VMEM((32,), jnp.float32),
            pltpu.VMEM((kpad,), jnp.int32),
            pltpu.SemaphoreType.DMA,
        ],
    )(sums_nc, lanes)

    out = pl.pallas_call(
        lambda i_ref, x_ref, o_ref, z_ref, sem: _zero_kernel(
            n * k, c, k, i_ref, x_ref, o_ref, z_ref, sem),
        grid_spec=pltpu.PrefetchScalarGridSpec(
            num_scalar_prefetch=1,
            grid=(1,),
            in_specs=[pl.BlockSpec(memory_space=pl.ANY)],
            out_specs=pl.BlockSpec(memory_space=pl.ANY),
            scratch_shapes=[
                pltpu.VMEM((1, 1, h, w), x.dtype),
                pltpu.SemaphoreType.DMA,
            ],
        ),
        out_shape=jax.ShapeDtypeStruct((n, c, h, w), x.dtype),
        input_output_aliases={1: 0},
    )(idx, copy)

    return out
